# baseline (speedup 1.0000x reference)
import functools

import jax
import jax.numpy as jnp
from jax.experimental import pallas as pl

B, W, R, D, E = 1024, 20, 10, 64, 256
G = 3 * D
BB = 256


def _interaction_kernel(ehT_ref, spkT_ref, adrT_ref,
                        wse_ref, wsd_ref, wsh_ref, bsi_ref, bsh_ref,
                        wae_ref, wad_ref, wah_ref, bai_ref, bah_ref,
                        woe_ref, woh_ref, boi_ref, boh_ref,
                        out_ref):
    f32 = jnp.float32
    eh2 = ehT_ref[...].reshape(W * BB, E)
    gis = (jnp.dot(eh2, wse_ref[...], preferred_element_type=f32)
           + bsi_ref[...]).reshape(W, BB, G)
    gia = (jnp.dot(eh2, wae_ref[...], preferred_element_type=f32)
           + bai_ref[...]).reshape(W, BB, G)
    gio = (jnp.dot(eh2, woe_ref[...], preferred_element_type=f32)
           + boi_ref[...]).reshape(W, BB, G)

    iota_rb = jax.lax.broadcasted_iota(jnp.int32, (R, BB), 0)
    A = jnp.zeros((R, BB, D), dtype=f32)

    wsd = wsd_ref[...]
    wsh = wsh_ref[...]
    bsh = bsh_ref[...]
    wad = wad_ref[...]
    wah = wah_ref[...]
    bah = bah_ref[...]
    woh = woh_ref[...]
    boh = boh_ref[...]

    def gru_tail(gi, gh, h):
        r = jax.nn.sigmoid(gi[..., :D] + gh[..., :D])
        z = jax.nn.sigmoid(gi[..., D:2 * D] + gh[..., D:2 * D])
        n = jnp.tanh(gi[..., 2 * D:] + r * gh[..., 2 * D:])
        return (1.0 - z) * n + z * h

    for T in range(W):
        m_spk = (iota_rb == spkT_ref[T:T + 1, :]).astype(f32)
        m_adr = (iota_rb == adrT_ref[T:T + 1, :]).astype(f32)
        spk_v = jnp.sum(A * m_spk[:, :, None], axis=0)
        adr_v = jnp.sum(A * m_adr[:, :, None], axis=0)

        gi_s = gis[T] + jnp.dot(adr_v, wsd, preferred_element_type=f32)
        gh_s = jnp.dot(spk_v, wsh, preferred_element_type=f32) + bsh
        new_spk = gru_tail(gi_s, gh_s, spk_v)

        gi_a = gia[T] + jnp.dot(spk_v, wad, preferred_element_type=f32)
        gh_a = jnp.dot(adr_v, wah, preferred_element_type=f32) + bah
        new_adr = gru_tail(gi_a, gh_a, adr_v)

        gh_o = (jnp.dot(A.reshape(R * BB, D), woh, preferred_element_type=f32)
                + boh).reshape(R, BB, G)
        new_oth = gru_tail(gio[T][None, :, :], gh_o, A)

        m_oth = 1.0 - m_spk - m_adr
        A = (m_oth[:, :, None] * new_oth
             + m_spk[:, :, None] * new_spk[None, :, :]
             + m_adr[:, :, None] * new_adr[None, :, :])

    out_ref[...] = A


@functools.partial(jax.jit, static_argnames=("interpret",))
def kernel(encoder_hiddens, dig_users, Ws_ih, Ws_hh, bs_ih, bs_hh,
           Wa_ih, Wa_hh, ba_ih, ba_hh, Wo_ih, Wo_hh, bo_ih, bo_hh,
           interpret=False):
    ehT = jnp.transpose(encoder_hiddens, (1, 0, 2))
    spkT = dig_users[:, :, 0].T.astype(jnp.int32)
    adrT = dig_users[:, :, 1].T.astype(jnp.int32)
    wse = Ws_ih[:, :E].T
    wsd = Ws_ih[:, E:].T
    wsh = Ws_hh.T
    wae = Wa_ih[:, :E].T
    wad = Wa_ih[:, E:].T
    wah = Wa_hh.T
    woe = Wo_ih.T
    woh = Wo_hh.T
    bsi = bs_ih.reshape(1, G)
    bsh = bs_hh.reshape(1, G)
    bai = ba_ih.reshape(1, G)
    bah = ba_hh.reshape(1, G)
    boi = bo_ih.reshape(1, G)
    boh = bo_hh.reshape(1, G)

    nb = B // BB
    bspec = lambda shape: pl.BlockSpec(shape, lambda i: (0,) * len(shape))
    grid_spec = pl.GridSpec(
        grid=(nb,),
        in_specs=[
            pl.BlockSpec((W, BB, E), lambda i: (0, i, 0)),
            pl.BlockSpec((W, BB), lambda i: (0, i)),
            pl.BlockSpec((W, BB), lambda i: (0, i)),
            bspec((E, G)), bspec((D, G)), bspec((D, G)), bspec((1, G)), bspec((1, G)),
            bspec((E, G)), bspec((D, G)), bspec((D, G)), bspec((1, G)), bspec((1, G)),
            bspec((E, G)), bspec((D, G)), bspec((1, G)), bspec((1, G)),
        ],
        out_specs=pl.BlockSpec((R, BB, D), lambda i: (0, i, 0)),
    )
    out = pl.pallas_call(
        _interaction_kernel,
        grid_spec=grid_spec,
        out_shape=jax.ShapeDtypeStruct((R, B, D), jnp.float32),
        interpret=interpret,
    )(ehT, spkT, adrT,
      wse, wsd, wsh, bsi, bsh,
      wae, wad, wah, bai, bah,
      woe, woh, boi, boh)
    return jnp.transpose(out, (1, 0, 2))


# speedup vs baseline: 1.0508x; 1.0508x over previous
import functools

import jax
import jax.numpy as jnp
from jax.experimental import pallas as pl

B, W, R, D, E = 1024, 20, 10, 64, 256
G = 3 * D
BB = 512


def _interaction_kernel(ehT_ref, spkT_ref, adrT_ref,
                        wse_ref, wsd_ref, wsh_ref, bsi_ref, bsh_ref,
                        wae_ref, wad_ref, wah_ref, bai_ref, bah_ref,
                        woe_ref, woh_ref, boi_ref, boh_ref,
                        out_ref):
    f32 = jnp.float32
    eh2 = ehT_ref[...].reshape(W * BB, E)
    gis = (jnp.dot(eh2, wse_ref[...], preferred_element_type=f32)
           + bsi_ref[...]).reshape(W, BB, G)
    gia = (jnp.dot(eh2, wae_ref[...], preferred_element_type=f32)
           + bai_ref[...]).reshape(W, BB, G)
    gio = (jnp.dot(eh2, woe_ref[...], preferred_element_type=f32)
           + boi_ref[...]).reshape(W, BB, G)

    iota_rb = jax.lax.broadcasted_iota(jnp.int32, (R, BB), 0)
    A = jnp.zeros((R, BB, D), dtype=f32)

    wsd = wsd_ref[...]
    wsh = wsh_ref[...]
    bsh = bsh_ref[...]
    wad = wad_ref[...]
    wah = wah_ref[...]
    bah = bah_ref[...]
    woh = woh_ref[...]
    boh = boh_ref[...]

    def gru_tail(gi, gh, h):
        r = jax.nn.sigmoid(gi[..., :D] + gh[..., :D])
        z = jax.nn.sigmoid(gi[..., D:2 * D] + gh[..., D:2 * D])
        n = jnp.tanh(gi[..., 2 * D:] + r * gh[..., 2 * D:])
        return (1.0 - z) * n + z * h

    for T in range(W):
        m_spk = (iota_rb == spkT_ref[T:T + 1, :]).astype(f32)
        m_adr = (iota_rb == adrT_ref[T:T + 1, :]).astype(f32)
        spk_v = jnp.sum(A * m_spk[:, :, None], axis=0)
        adr_v = jnp.sum(A * m_adr[:, :, None], axis=0)

        gi_s = gis[T] + jnp.dot(adr_v, wsd, preferred_element_type=f32)
        gh_s = jnp.dot(spk_v, wsh, preferred_element_type=f32) + bsh
        new_spk = gru_tail(gi_s, gh_s, spk_v)

        gi_a = gia[T] + jnp.dot(spk_v, wad, preferred_element_type=f32)
        gh_a = jnp.dot(adr_v, wah, preferred_element_type=f32) + bah
        new_adr = gru_tail(gi_a, gh_a, adr_v)

        gh_o = (jnp.dot(A.reshape(R * BB, D), woh, preferred_element_type=f32)
                + boh).reshape(R, BB, G)
        new_oth = gru_tail(gio[T][None, :, :], gh_o, A)

        m_oth = 1.0 - m_spk - m_adr
        A = (m_oth[:, :, None] * new_oth
             + m_spk[:, :, None] * new_spk[None, :, :]
             + m_adr[:, :, None] * new_adr[None, :, :])

    out_ref[...] = A


@functools.partial(jax.jit, static_argnames=("interpret",))
def kernel(encoder_hiddens, dig_users, Ws_ih, Ws_hh, bs_ih, bs_hh,
           Wa_ih, Wa_hh, ba_ih, ba_hh, Wo_ih, Wo_hh, bo_ih, bo_hh,
           interpret=False):
    ehT = jnp.transpose(encoder_hiddens, (1, 0, 2))
    spkT = dig_users[:, :, 0].T.astype(jnp.int32)
    adrT = dig_users[:, :, 1].T.astype(jnp.int32)
    wse = Ws_ih[:, :E].T.astype(jnp.bfloat16)
    wsd = Ws_ih[:, E:].T
    wsh = Ws_hh.T
    wae = Wa_ih[:, :E].T.astype(jnp.bfloat16)
    wad = Wa_ih[:, E:].T
    wah = Wa_hh.T
    woe = Wo_ih.T.astype(jnp.bfloat16)
    woh = Wo_hh.T
    bsi = bs_ih.reshape(1, G)
    bsh = bs_hh.reshape(1, G)
    bai = ba_ih.reshape(1, G)
    bah = ba_hh.reshape(1, G)
    boi = bo_ih.reshape(1, G)
    boh = bo_hh.reshape(1, G)

    nb = B // BB
    bspec = lambda shape: pl.BlockSpec(shape, lambda i: (0,) * len(shape))
    grid_spec = pl.GridSpec(
        grid=(nb,),
        in_specs=[
            pl.BlockSpec((W, BB, E), lambda i: (0, i, 0)),
            pl.BlockSpec((W, BB), lambda i: (0, i)),
            pl.BlockSpec((W, BB), lambda i: (0, i)),
            bspec((E, G)), bspec((D, G)), bspec((D, G)), bspec((1, G)), bspec((1, G)),
            bspec((E, G)), bspec((D, G)), bspec((D, G)), bspec((1, G)), bspec((1, G)),
            bspec((E, G)), bspec((D, G)), bspec((1, G)), bspec((1, G)),
        ],
        out_specs=pl.BlockSpec((R, BB, D), lambda i: (0, i, 0)),
    )
    out = pl.pallas_call(
        _interaction_kernel,
        grid_spec=grid_spec,
        out_shape=jax.ShapeDtypeStruct((R, B, D), jnp.float32),
        interpret=interpret,
    )(ehT.astype(jnp.bfloat16), spkT, adrT,
      wse, wsd, wsh, bsi, bsh,
      wae, wad, wah, bai, bah,
      woe, woh, boi, boh)
    return jnp.transpose(out, (1, 0, 2))
